# baseline (speedup 1.0000x reference)
import functools

import jax
import jax.numpy as jnp
from jax import lax
from jax.experimental import pallas as pl
from jax.experimental.pallas import tpu as pltpu
from jax.experimental.pallas import tpu_sc as plsc

B, T, TT, F, K = 4, 4096, 4, 128, 65536
N = T + TT
NC, NS, L = 2, 16, 16
NW = NC * NS
CHUNK = K // NS
NVEC = CHUNK // L
WWIN = CHUNK + 16
BPC = B // NC


def _sc_body(src_h, dst_h, w_h, nodes_h, part_h, cnt_h,
             src_v, dst_v, comp_src, comp_rank, comp_tau,
             wwin, rows_v, rows_b, acc_v, cnt_buf, base_buf, sem):
    c = lax.axis_index("c")
    s = lax.axis_index("s")
    wid = c * NS + s
    iota = lax.iota(jnp.int32, L)
    zf = jnp.zeros((L,), jnp.float32)

    def zero_body(i, _):
        acc_v[pl.ds(i * L, L)] = zf
        return _
    lax.fori_loop(0, B * TT * F // L, zero_body, 0)

    for ib in range(BPC):
        b = c + NC * ib
        base_edge = pl.multiple_of(b * K + s * CHUNK, 8)
        pltpu.sync_copy(src_h.at[pl.ds(base_edge, CHUNK)], src_v)
        pltpu.sync_copy(dst_h.at[pl.ds(base_edge, CHUNK)], dst_v)

        onei = jnp.ones((L,), jnp.int32)
        zeroi = jnp.zeros((L,), jnp.int32)

        pack = jnp.full((L,), 0x01010101, jnp.int32)

        def scan_body(i, carry):
            ncomp, c0, c1, c2, c3 = carry
            sv = src_v[pl.ds(i * L, L)]
            dv = dst_v[pl.ds(i * L, L)]
            u = jnp.clip(jnp.maximum(sv, dv) - T, 0, TT)
            e = jnp.where(u < TT, pack << (jnp.clip(u, 0, TT - 1) << 3), zeroi)
            ic = plsc.cumsum(e)
            exc = ic - e
            tot = jnp.sum(e)
            cm = (dv >= T) & (sv <= dv)
            tauv = dv - T
            csel = jnp.where(tauv == 1, jnp.broadcast_to(c1, (L,)),
                    jnp.where(tauv == 2, jnp.broadcast_to(c2, (L,)),
                     jnp.where(tauv == 3, jnp.broadcast_to(c3, (L,)),
                               jnp.broadcast_to(c0, (L,)))))
            rank = csel + ((exc >> (jnp.clip(tauv, 0, TT - 1) << 3)) & 0xFF)
            plsc.store_compressed(comp_src.at[pl.ds(ncomp, L)], sv, mask=cm)
            plsc.store_compressed(comp_rank.at[pl.ds(ncomp, L)], rank, mask=cm)
            plsc.store_compressed(comp_tau.at[pl.ds(ncomp, L)], tauv, mask=cm)
            ncomp = ncomp + jnp.sum(jnp.where(cm, onei, zeroi))
            return (ncomp, c0 + (tot & 0xFF), c1 + ((tot >> 8) & 0xFF),
                    c2 + ((tot >> 16) & 0xFF), c3 + ((tot >> 24) & 0xFF))

        z = jnp.int32(0)
        ncomp, c0, c1, c2, c3 = lax.fori_loop(
            0, NVEC, scan_body, (z, z, z, z, z))

        cntv = jnp.zeros((L,), jnp.int32)
        for t, ct in enumerate((c0, c1, c2, c3)):
            cntv = jnp.where(iota == t, jnp.broadcast_to(ct, (L,)), cntv)
        cnt_buf[...] = cntv
        pltpu.sync_copy(cnt_buf, cnt_h.at[c, s])
        plsc.subcore_barrier()
        pltpu.sync_copy(cnt_h.at[c], base_buf)
        plsc.subcore_barrier()
        basev = jnp.zeros((L,), jnp.int32)
        sbc = jnp.broadcast_to(s, (L,))
        for sp in range(NS):
            maskv = jnp.where(jnp.broadcast_to(sp, (L,)) < sbc, onei, zeroi)
            basev = basev + base_buf[sp] * maskv
        base = [jnp.sum(jnp.where(iota == t, basev, zeroi))
                for t in range(TT)]

        woff = []
        for t in range(TT):
            start8 = (base[t] >> 3) << 3
            wstart = pl.multiple_of(b * K + start8, 8)
            pltpu.sync_copy(w_h.at[pl.ds(wstart, WWIN)],
                            wwin.at[pl.ds(t * WWIN, WWIN)])
            woff.append(base[t] - start8 + t * WWIN)

        acc_base = b * TT * F
        ngroups = (ncomp + (L - 1)) // L

        wsel0 = jnp.broadcast_to(woff[0], (L,))
        wsel1 = jnp.broadcast_to(woff[1], (L,))
        wsel2 = jnp.broadcast_to(woff[2], (L,))
        wsel3 = jnp.broadcast_to(woff[3], (L,))

        def do_group(g, rbuf):
            off = g * L
            sv = comp_src[pl.ds(off, L)]
            rv = comp_rank[pl.ds(off, L)]
            tv = comp_tau[pl.ds(off, L)]
            valid = (off + iota) < jnp.broadcast_to(ncomp, (L,))
            tvc = jnp.clip(tv, 0, TT - 1)
            wsel = jnp.where(tvc == 0, wsel0,
                    jnp.where(tvc == 1, wsel1,
                     jnp.where(tvc == 2, wsel2, wsel3)))
            widx = jnp.clip(rv + wsel, 0, TT * WWIN - 1)
            wv = plsc.load_gather(wwin, [widx])
            wv = jnp.where(valid, wv, zf)
            rowidx = jnp.clip(jnp.broadcast_to(b * N, (L,)) + sv, 0, B * N - 1)
            pltpu.async_copy(nodes_h.at[rowidx], rbuf, sem).wait()
            for lane in range(L):
                wl = jnp.sum(jnp.where(iota == lane, wv, zf))
                tl = jnp.sum(jnp.where(iota == lane, tvc, zeroi))
                aoff = acc_base + tl * F
                for k2 in range(F // L):
                    acc_v[pl.ds(aoff + k2 * L, L)] = (
                        acc_v[pl.ds(aoff + k2 * L, L)]
                        + wl * rbuf[lane, pl.ds(k2 * L, L)])

        def group_body(g, _):
            even = (g & 1) == 0

            @pl.when(even)
            def _a():
                do_group(g, rows_v)

            @pl.when(jnp.logical_not(even))
            def _b():
                do_group(g, rows_b)
            return _

        lax.fori_loop(0, ngroups, group_body, 0)

    pltpu.sync_copy(acc_v, part_h.at[wid])


@functools.partial(
    pl.kernel,
    out_type=(jax.ShapeDtypeStruct((NW, B * TT * F), jnp.float32),
              jax.ShapeDtypeStruct((NC, NS, L), jnp.int32)),
    mesh=plsc.VectorSubcoreMesh(core_axis_name="c", subcore_axis_name="s",
                                num_cores=NC, num_subcores=NS),
    compiler_params=pltpu.CompilerParams(needs_layout_passes=False),
    scratch_types=[
        pltpu.VMEM((CHUNK,), jnp.int32),
        pltpu.VMEM((CHUNK,), jnp.int32),
        pltpu.VMEM((CHUNK + L,), jnp.int32),
        pltpu.VMEM((CHUNK + L,), jnp.int32),
        pltpu.VMEM((CHUNK + L,), jnp.int32),
        pltpu.VMEM((TT * WWIN,), jnp.float32),
        pltpu.VMEM((L, F), jnp.float32),
        pltpu.VMEM((L, F), jnp.float32),
        pltpu.VMEM((B * TT * F,), jnp.float32),
        pltpu.VMEM((L,), jnp.int32),
        pltpu.VMEM((NS, L), jnp.int32),
        pltpu.SemaphoreType.DMA,
    ],
)
def _sc_edges(src_h, dst_h, w_h, nodes_h, part_h, cnt_h, *scratch):
    _sc_body(src_h, dst_h, w_h, nodes_h, part_h, cnt_h, *scratch)


def _tc_body(part_ref, x_ref, wm_ref, wr_ref, bias_ref, out_ref):
    u = jnp.sum(part_ref[...], axis=0)
    out_ref[...] = (
        jnp.dot(u, wm_ref[...], preferred_element_type=jnp.float32)
        + jnp.dot(x_ref[...], wr_ref[...], preferred_element_type=jnp.float32)
        + bias_ref[...])


def _tc_finish(part, x2d, W_msg, W_root, bias2d):
    return pl.pallas_call(
        _tc_body,
        out_shape=jax.ShapeDtypeStruct((B * TT, F), jnp.float32),
    )(part, x2d, W_msg, W_root, bias2d)


def kernel(x, nodes, edge_list, weights, W_msg, W_root, bias):
    nodes_full = jnp.concatenate((nodes, x), axis=1)
    src = edge_list[:, 0, :].reshape(B * K)
    dst = edge_list[:, 1, :].reshape(B * K)
    wflat = jnp.pad(weights.reshape(B * K), (0, WWIN))
    part, _ = _sc_edges(src, dst, wflat, nodes_full.reshape(B * N, F))
    mx = _tc_finish(part.reshape(NW, B * TT, F),
                    x.reshape(B * TT, F), W_msg, W_root, bias.reshape(1, F))
    return (mx.reshape(B, TT, F), nodes_full, edge_list, weights)


# speedup vs baseline: 1.0770x; 1.0770x over previous
import functools

import jax
import jax.numpy as jnp
from jax import lax
from jax.experimental import pallas as pl
from jax.experimental.pallas import tpu as pltpu
from jax.experimental.pallas import tpu_sc as plsc

B, T, TT, F, K = 4, 4096, 4, 128, 65536
N = T + TT
NC, NS, L = 2, 16, 16
NW = NC * NS
CHUNK = K // NS
NVEC = CHUNK // L
WWIN = CHUNK + 16
BPC = B // NC


def _sc_body(src_h, dst_h, w_h, nodes_in, x_in, part_h, cnt_h,
             src_v, dst_v, comp_src, comp_rank, comp_tau,
             wwin, rows_v, rows_b, x_v, acc_v, cnt_buf, base_buf, sem):
    c = lax.axis_index("c")
    s = lax.axis_index("s")
    wid = c * NS + s
    iota = lax.iota(jnp.int32, L)
    zf = jnp.zeros((L,), jnp.float32)

    def zero_body(i, _):
        acc_v[pl.ds(i * L, L)] = zf
        return _
    lax.fori_loop(0, B * TT * F // L, zero_body, 0)

    pltpu.sync_copy(x_in, x_v)

    for ib in range(BPC):
        b = c + NC * ib
        base_edge = pl.multiple_of(b * K + s * CHUNK, 8)
        pltpu.sync_copy(src_h.at[pl.ds(base_edge, CHUNK)], src_v)
        pltpu.sync_copy(dst_h.at[pl.ds(base_edge, CHUNK)], dst_v)

        onei = jnp.ones((L,), jnp.int32)
        zeroi = jnp.zeros((L,), jnp.int32)

        pack = jnp.full((L,), 0x01010101, jnp.int32)

        def scan_body(i, carry):
            ncomp, c0, c1, c2, c3 = carry
            sv = src_v[pl.ds(i * L, L)]
            dv = dst_v[pl.ds(i * L, L)]
            u = jnp.clip(jnp.maximum(sv, dv) - T, 0, TT)
            e = jnp.where(u < TT, pack << (jnp.clip(u, 0, TT - 1) << 3), zeroi)
            ic = plsc.cumsum(e)
            exc = ic - e
            tot = jnp.sum(e)
            cm = (dv >= T) & (sv <= dv)
            tauv = dv - T
            csel = jnp.where(tauv == 1, jnp.broadcast_to(c1, (L,)),
                    jnp.where(tauv == 2, jnp.broadcast_to(c2, (L,)),
                     jnp.where(tauv == 3, jnp.broadcast_to(c3, (L,)),
                               jnp.broadcast_to(c0, (L,)))))
            rank = csel + ((exc >> (jnp.clip(tauv, 0, TT - 1) << 3)) & 0xFF)
            plsc.store_compressed(comp_src.at[pl.ds(ncomp, L)], sv, mask=cm)
            plsc.store_compressed(comp_rank.at[pl.ds(ncomp, L)], rank, mask=cm)
            plsc.store_compressed(comp_tau.at[pl.ds(ncomp, L)], tauv, mask=cm)
            ncomp = ncomp + jnp.sum(jnp.where(cm, onei, zeroi))
            return (ncomp, c0 + (tot & 0xFF), c1 + ((tot >> 8) & 0xFF),
                    c2 + ((tot >> 16) & 0xFF), c3 + ((tot >> 24) & 0xFF))

        z = jnp.int32(0)
        ncomp, c0, c1, c2, c3 = lax.fori_loop(
            0, NVEC, scan_body, (z, z, z, z, z))

        cntv = jnp.zeros((L,), jnp.int32)
        for t, ct in enumerate((c0, c1, c2, c3)):
            cntv = jnp.where(iota == t, jnp.broadcast_to(ct, (L,)), cntv)
        cnt_buf[...] = cntv
        pltpu.sync_copy(cnt_buf, cnt_h.at[c, s])
        plsc.subcore_barrier()
        pltpu.sync_copy(cnt_h.at[c], base_buf)
        plsc.subcore_barrier()
        basev = jnp.zeros((L,), jnp.int32)
        sbc = jnp.broadcast_to(s, (L,))
        for sp in range(NS):
            maskv = jnp.where(jnp.broadcast_to(sp, (L,)) < sbc, onei, zeroi)
            basev = basev + base_buf[sp] * maskv
        base = [jnp.sum(jnp.where(iota == t, basev, zeroi))
                for t in range(TT)]

        woff = []
        for t in range(TT):
            start8 = (base[t] >> 3) << 3
            wstart = pl.multiple_of(b * K + start8, 8)
            pltpu.sync_copy(w_h.at[pl.ds(wstart, WWIN)],
                            wwin.at[pl.ds(t * WWIN, WWIN)])
            woff.append(base[t] - start8 + t * WWIN)

        acc_base = b * TT * F
        ngroups = (ncomp + (L - 1)) // L

        wsel0 = jnp.broadcast_to(woff[0], (L,))
        wsel1 = jnp.broadcast_to(woff[1], (L,))
        wsel2 = jnp.broadcast_to(woff[2], (L,))
        wsel3 = jnp.broadcast_to(woff[3], (L,))

        def do_group(g, rbuf):
            off = g * L
            sv = comp_src[pl.ds(off, L)]
            rv = comp_rank[pl.ds(off, L)]
            tv = comp_tau[pl.ds(off, L)]
            valid = (off + iota) < jnp.broadcast_to(ncomp, (L,))
            tvc = jnp.clip(tv, 0, TT - 1)
            wsel = jnp.where(tvc == 0, wsel0,
                    jnp.where(tvc == 1, wsel1,
                     jnp.where(tvc == 2, wsel2, wsel3)))
            widx = jnp.clip(rv + wsel, 0, TT * WWIN - 1)
            wv = plsc.load_gather(wwin, [widx])
            wv = jnp.where(valid, wv, zf)
            rowidx = jnp.clip(jnp.broadcast_to(b * T, (L,)) + sv, 0, B * T - 1)
            pltpu.async_copy(nodes_in.at[rowidx], rbuf, sem).wait()
            for lane in range(L):
                wl = jnp.sum(jnp.where(iota == lane, wv, zf))
                tl = jnp.sum(jnp.where(iota == lane, tvc, zeroi))
                sl = jnp.sum(jnp.where(iota == lane, sv, zeroi))
                wg = jnp.where(sl < T, wl, 0.0)
                wx = wl - wg
                xrow = b * TT + jnp.clip(sl - T, 0, TT - 1)
                aoff = acc_base + tl * F
                for k2 in range(F // L):
                    acc_v[pl.ds(aoff + k2 * L, L)] = (
                        acc_v[pl.ds(aoff + k2 * L, L)]
                        + wg * rbuf[lane, pl.ds(k2 * L, L)]
                        + wx * x_v[xrow, pl.ds(k2 * L, L)])

        def group_body(g, _):
            even = (g & 1) == 0

            @pl.when(even)
            def _a():
                do_group(g, rows_v)

            @pl.when(jnp.logical_not(even))
            def _b():
                do_group(g, rows_b)
            return _

        lax.fori_loop(0, ngroups, group_body, 0)

    pltpu.sync_copy(acc_v, part_h.at[wid])


@functools.partial(
    pl.kernel,
    out_type=(jax.ShapeDtypeStruct((NW, B * TT * F), jnp.float32),
              jax.ShapeDtypeStruct((NC, NS, L), jnp.int32)),
    mesh=plsc.VectorSubcoreMesh(core_axis_name="c", subcore_axis_name="s",
                                num_cores=NC, num_subcores=NS),
    compiler_params=pltpu.CompilerParams(needs_layout_passes=False),
    scratch_types=[
        pltpu.VMEM((CHUNK,), jnp.int32),
        pltpu.VMEM((CHUNK,), jnp.int32),
        pltpu.VMEM((CHUNK + L,), jnp.int32),
        pltpu.VMEM((CHUNK + L,), jnp.int32),
        pltpu.VMEM((CHUNK + L,), jnp.int32),
        pltpu.VMEM((TT * WWIN,), jnp.float32),
        pltpu.VMEM((L, F), jnp.float32),
        pltpu.VMEM((L, F), jnp.float32),
        pltpu.VMEM((B * TT, F), jnp.float32),
        pltpu.VMEM((B * TT * F,), jnp.float32),
        pltpu.VMEM((L,), jnp.int32),
        pltpu.VMEM((NS, L), jnp.int32),
        pltpu.SemaphoreType.DMA,
    ],
)
def _sc_edges(src_h, dst_h, w_h, nodes_in, x_in, part_h, cnt_h, *scratch):
    _sc_body(src_h, dst_h, w_h, nodes_in, x_in, part_h, cnt_h, *scratch)


def _concat_body(nodes_ref, x_ref, out_ref):
    out_ref[:, :T, :] = nodes_ref[...]
    out_ref[:, T:, :] = x_ref[...]


def _tc_concat(nodes, x):
    return pl.pallas_call(
        _concat_body,
        out_shape=jax.ShapeDtypeStruct((B, N, F), jnp.float32),
    )(nodes, x)


def _tc_body(part_ref, x_ref, wm_ref, wr_ref, bias_ref, out_ref):
    u = jnp.sum(part_ref[...], axis=0)
    out_ref[...] = (
        jnp.dot(u, wm_ref[...], preferred_element_type=jnp.float32)
        + jnp.dot(x_ref[...], wr_ref[...], preferred_element_type=jnp.float32)
        + bias_ref[...])


def _tc_finish(part, x2d, W_msg, W_root, bias2d):
    return pl.pallas_call(
        _tc_body,
        out_shape=jax.ShapeDtypeStruct((B * TT, F), jnp.float32),
    )(part, x2d, W_msg, W_root, bias2d)


def kernel(x, nodes, edge_list, weights, W_msg, W_root, bias):
    src = edge_list[:, 0, :].reshape(B * K)
    dst = edge_list[:, 1, :].reshape(B * K)
    wflat = jnp.pad(weights.reshape(B * K), (0, WWIN))
    part, _ = _sc_edges(src, dst, wflat, nodes.reshape(B * T, F),
                        x.reshape(B * TT, F))
    nodes_full = _tc_concat(nodes, x)
    mx = _tc_finish(part.reshape(NW, B * TT, F),
                    x.reshape(B * TT, F), W_msg, W_root, bias.reshape(1, F))
    return (mx.reshape(B, TT, F), nodes_full, edge_list, weights)


# speedup vs baseline: 1.0905x; 1.0125x over previous
import functools

import jax
import jax.numpy as jnp
from jax import lax
from jax.experimental import pallas as pl
from jax.experimental.pallas import tpu as pltpu
from jax.experimental.pallas import tpu_sc as plsc

B, T, TT, F, K = 4, 4096, 4, 128, 65536
N = T + TT
NC, NS, L = 2, 16, 16
NW = NC * NS
CHUNK = K // NS
NVEC = CHUNK // L
WWIN = CHUNK + 16
BPC = B // NC


def _sc_body(src_h, dst_h, w_h, nodes_in, x_in, part_h, cnt_h,
             src_v, dst_v, comp_src, comp_rank, comp_tau,
             wwin, rows_v, rows_b, x_v, acc_v, cnt_buf, base_buf, sem):
    c = lax.axis_index("c")
    s = lax.axis_index("s")
    wid = c * NS + s
    iota = lax.iota(jnp.int32, L)
    zf = jnp.zeros((L,), jnp.float32)

    def zero_body(i, _):
        acc_v[pl.ds(i * L, L)] = zf
        return _
    lax.fori_loop(0, B * TT * F // L, zero_body, 0)

    pltpu.sync_copy(x_in, x_v)

    for ib in range(BPC):
        b = c + NC * ib
        base_edge = pl.multiple_of(b * K + s * CHUNK, 8)
        pltpu.sync_copy(src_h.at[pl.ds(base_edge, CHUNK)], src_v)
        pltpu.sync_copy(dst_h.at[pl.ds(base_edge, CHUNK)], dst_v)

        onei = jnp.ones((L,), jnp.int32)
        zeroi = jnp.zeros((L,), jnp.int32)

        pack = jnp.full((L,), 0x01010101, jnp.int32)

        def scan_step(off, carry):
            ncomp, c0, c1, c2, c3 = carry
            sv = src_v[pl.ds(off, L)]
            dv = dst_v[pl.ds(off, L)]
            u = jnp.clip(jnp.maximum(sv, dv) - T, 0, TT)
            e = jnp.where(u < TT, pack << (jnp.clip(u, 0, TT - 1) << 3), zeroi)
            ic = plsc.cumsum(e)
            exc = ic - e
            tot = jnp.sum(e)
            cm = (dv >= T) & (sv <= dv)
            tauv = dv - T
            csel = jnp.where(tauv == 1, jnp.broadcast_to(c1, (L,)),
                    jnp.where(tauv == 2, jnp.broadcast_to(c2, (L,)),
                     jnp.where(tauv == 3, jnp.broadcast_to(c3, (L,)),
                               jnp.broadcast_to(c0, (L,)))))
            rank = csel + ((exc >> (jnp.clip(tauv, 0, TT - 1) << 3)) & 0xFF)
            plsc.store_compressed(comp_src.at[pl.ds(ncomp, L)], sv, mask=cm)
            plsc.store_compressed(comp_rank.at[pl.ds(ncomp, L)], rank, mask=cm)
            plsc.store_compressed(comp_tau.at[pl.ds(ncomp, L)], tauv, mask=cm)
            ncomp = ncomp + jnp.sum(jnp.where(cm, onei, zeroi))
            return (ncomp, c0 + (tot & 0xFF), c1 + ((tot >> 8) & 0xFF),
                    c2 + ((tot >> 16) & 0xFF), c3 + ((tot >> 24) & 0xFF))

        def scan_body(i, carry):
            carry = scan_step(i * (2 * L), carry)
            return scan_step(i * (2 * L) + L, carry)

        z = jnp.int32(0)
        ncomp, c0, c1, c2, c3 = lax.fori_loop(
            0, NVEC // 2, scan_body, (z, z, z, z, z))

        cntv = jnp.zeros((L,), jnp.int32)
        for t, ct in enumerate((c0, c1, c2, c3)):
            cntv = jnp.where(iota == t, jnp.broadcast_to(ct, (L,)), cntv)
        cnt_buf[...] = cntv
        pltpu.sync_copy(cnt_buf, cnt_h.at[c, s])
        plsc.subcore_barrier()
        pltpu.sync_copy(cnt_h.at[c], base_buf)
        plsc.subcore_barrier()
        basev = jnp.zeros((L,), jnp.int32)
        sbc = jnp.broadcast_to(s, (L,))
        for sp in range(NS):
            maskv = jnp.where(jnp.broadcast_to(sp, (L,)) < sbc, onei, zeroi)
            basev = basev + base_buf[sp] * maskv
        base = [jnp.sum(jnp.where(iota == t, basev, zeroi))
                for t in range(TT)]

        woff = []
        for t in range(TT):
            start8 = (base[t] >> 3) << 3
            wstart = pl.multiple_of(b * K + start8, 8)
            pltpu.sync_copy(w_h.at[pl.ds(wstart, WWIN)],
                            wwin.at[pl.ds(t * WWIN, WWIN)])
            woff.append(base[t] - start8 + t * WWIN)

        acc_base = b * TT * F
        ngroups = (ncomp + (L - 1)) // L

        wsel0 = jnp.broadcast_to(woff[0], (L,))
        wsel1 = jnp.broadcast_to(woff[1], (L,))
        wsel2 = jnp.broadcast_to(woff[2], (L,))
        wsel3 = jnp.broadcast_to(woff[3], (L,))

        def do_group(g, rbuf):
            off = g * L
            sv = comp_src[pl.ds(off, L)]
            rv = comp_rank[pl.ds(off, L)]
            tv = comp_tau[pl.ds(off, L)]
            valid = (off + iota) < jnp.broadcast_to(ncomp, (L,))
            tvc = jnp.clip(tv, 0, TT - 1)
            wsel = jnp.where(tvc == 0, wsel0,
                    jnp.where(tvc == 1, wsel1,
                     jnp.where(tvc == 2, wsel2, wsel3)))
            widx = jnp.clip(rv + wsel, 0, TT * WWIN - 1)
            wv = plsc.load_gather(wwin, [widx])
            wv = jnp.where(valid, wv, zf)
            rowidx = jnp.clip(jnp.broadcast_to(b * T, (L,)) + sv, 0, B * T - 1)
            pltpu.async_copy(nodes_in.at[rowidx], rbuf, sem).wait()
            for lane in range(L):
                wl = jnp.sum(jnp.where(iota == lane, wv, zf))
                tl = jnp.sum(jnp.where(iota == lane, tvc, zeroi))
                sl = jnp.sum(jnp.where(iota == lane, sv, zeroi))
                wg = jnp.where(sl < T, wl, 0.0)
                wx = wl - wg
                xrow = b * TT + jnp.clip(sl - T, 0, TT - 1)
                aoff = acc_base + tl * F
                for k2 in range(F // L):
                    acc_v[pl.ds(aoff + k2 * L, L)] = (
                        acc_v[pl.ds(aoff + k2 * L, L)]
                        + wg * rbuf[lane, pl.ds(k2 * L, L)]
                        + wx * x_v[xrow, pl.ds(k2 * L, L)])

        def group_body(g, _):
            even = (g & 1) == 0

            @pl.when(even)
            def _a():
                do_group(g, rows_v)

            @pl.when(jnp.logical_not(even))
            def _b():
                do_group(g, rows_b)
            return _

        lax.fori_loop(0, ngroups, group_body, 0)

    pltpu.sync_copy(acc_v, part_h.at[wid])


@functools.partial(
    pl.kernel,
    out_type=(jax.ShapeDtypeStruct((NW, B * TT * F), jnp.float32),
              jax.ShapeDtypeStruct((NC, NS, L), jnp.int32)),
    mesh=plsc.VectorSubcoreMesh(core_axis_name="c", subcore_axis_name="s",
                                num_cores=NC, num_subcores=NS),
    compiler_params=pltpu.CompilerParams(needs_layout_passes=False),
    scratch_types=[
        pltpu.VMEM((CHUNK,), jnp.int32),
        pltpu.VMEM((CHUNK,), jnp.int32),
        pltpu.VMEM((CHUNK + L,), jnp.int32),
        pltpu.VMEM((CHUNK + L,), jnp.int32),
        pltpu.VMEM((CHUNK + L,), jnp.int32),
        pltpu.VMEM((TT * WWIN,), jnp.float32),
        pltpu.VMEM((L, F), jnp.float32),
        pltpu.VMEM((L, F), jnp.float32),
        pltpu.VMEM((B * TT, F), jnp.float32),
        pltpu.VMEM((B * TT * F,), jnp.float32),
        pltpu.VMEM((L,), jnp.int32),
        pltpu.VMEM((NS, L), jnp.int32),
        pltpu.SemaphoreType.DMA,
    ],
)
def _sc_edges(src_h, dst_h, w_h, nodes_in, x_in, part_h, cnt_h, *scratch):
    _sc_body(src_h, dst_h, w_h, nodes_in, x_in, part_h, cnt_h, *scratch)


def _concat_body(nodes_ref, x_ref, out_ref):
    out_ref[:, :T, :] = nodes_ref[...]
    out_ref[:, T:, :] = x_ref[...]


def _tc_concat(nodes, x):
    return pl.pallas_call(
        _concat_body,
        out_shape=jax.ShapeDtypeStruct((B, N, F), jnp.float32),
    )(nodes, x)


def _tc_body(part_ref, x_ref, wm_ref, wr_ref, bias_ref, out_ref):
    u = jnp.sum(part_ref[...], axis=0)
    out_ref[...] = (
        jnp.dot(u, wm_ref[...], preferred_element_type=jnp.float32)
        + jnp.dot(x_ref[...], wr_ref[...], preferred_element_type=jnp.float32)
        + bias_ref[...])


def _tc_finish(part, x2d, W_msg, W_root, bias2d):
    return pl.pallas_call(
        _tc_body,
        out_shape=jax.ShapeDtypeStruct((B * TT, F), jnp.float32),
    )(part, x2d, W_msg, W_root, bias2d)


def kernel(x, nodes, edge_list, weights, W_msg, W_root, bias):
    src = edge_list[:, 0, :].reshape(B * K)
    dst = edge_list[:, 1, :].reshape(B * K)
    wflat = jnp.pad(weights.reshape(B * K), (0, WWIN))
    part, _ = _sc_edges(src, dst, wflat, nodes.reshape(B * T, F),
                        x.reshape(B * TT, F))
    nodes_full = _tc_concat(nodes, x)
    mx = _tc_finish(part.reshape(NW, B * TT, F),
                    x.reshape(B * TT, F), W_msg, W_root, bias.reshape(1, F))
    return (mx.reshape(B, TT, F), nodes_full, edge_list, weights)
